# baseline (speedup 1.0000x reference)
import jax
import jax.numpy as jnp
from jax.experimental import pallas as pl
from jax.experimental.pallas import tpu as pltpu

WIDTH = 128
N_REL = 16
N_NODE = 512


def _body(x_ref, db_hbm, a1w_ref, a2w_ref, opw_ref, cw_ref,
          out_ref, dbv, sems):
    copies = [
        pltpu.make_async_copy(db_hbm.at[i], dbv.at[i], sems.at[i])
        for i in range(2 * N_REL)
    ]
    for c in copies:
        c.start()

    x = x_ref[...]
    w1 = a1w_ref[...]
    w1 = jnp.exp(w1 - jnp.max(w1, axis=0, keepdims=True))
    w1 = w1 / jnp.sum(w1, axis=0, keepdims=True)
    w2 = a2w_ref[...]
    w2 = jnp.exp(w2 - jnp.max(w2, axis=0, keepdims=True))
    w2 = w2 / jnp.sum(w2, axis=0, keepdims=True)
    arg1 = jax.lax.dot_general(
        w1, x, (((0,), (0,)), ((), ())), preferred_element_type=jnp.float32)
    arg2 = jax.lax.dot_general(
        w2, x, (((0,), (0,)), ((), ())), preferred_element_type=jnp.float32)
    cw = cw_ref[...]
    cw = jnp.exp(cw - jnp.max(cw, axis=1, keepdims=True))
    cwsm = cw / jnp.sum(cw, axis=1, keepdims=True)

    x2b = arg2.astype(jnp.bfloat16)
    acc = jnp.zeros((WIDTH, N_NODE), jnp.float32)
    for i in range(N_REL):
        copies[2 * i].wait()
        copies[2 * i + 1].wait()
        d = dbv[2 * i:2 * i + 2].reshape(N_NODE, N_NODE).astype(jnp.bfloat16)
        fwd = jax.lax.dot_general(
            x2b, d, (((1,), (0,)), ((), ())), preferred_element_type=jnp.float32)
        bwd = jax.lax.dot_general(
            x2b, d, (((1,), (1,)), ((), ())), preferred_element_type=jnp.float32)
        w1c = cwsm[:, i:i + 1]
        w2c = cwsm[:, N_REL + i:N_REL + i + 1]
        acc = acc + w1c * fwd + w2c * bwd

    chain = 1.0 - jnp.exp(-acc)
    opw = opw_ref[...]
    opw = jnp.exp(opw - jnp.max(opw, axis=1, keepdims=True))
    opw = opw / jnp.sum(opw, axis=1, keepdims=True)
    a12 = arg1 * arg2
    out_ref[...] = (opw[:, 0:1] * arg2
                    + opw[:, 1:2] * a12
                    + opw[:, 2:3] * (arg1 + arg2 - a12)
                    + opw[:, 3:4] * chain
                    + opw[:, 4:5] * (1.0 - arg1))


def kernel(inputs, database, arg1_weights, arg2_weights, op_weights, chain_weights):
    return pl.pallas_call(
        _body,
        in_specs=[
            pl.BlockSpec(memory_space=pltpu.MemorySpace.VMEM),
            pl.BlockSpec(memory_space=pltpu.MemorySpace.HBM),
            pl.BlockSpec(memory_space=pltpu.MemorySpace.VMEM),
            pl.BlockSpec(memory_space=pltpu.MemorySpace.VMEM),
            pl.BlockSpec(memory_space=pltpu.MemorySpace.VMEM),
            pl.BlockSpec(memory_space=pltpu.MemorySpace.VMEM),
        ],
        out_specs=pl.BlockSpec(memory_space=pltpu.MemorySpace.VMEM),
        out_shape=jax.ShapeDtypeStruct((WIDTH, N_NODE), jnp.float32),
        scratch_shapes=[
            pltpu.VMEM((2 * N_REL, N_NODE // 2, N_NODE), jnp.float32),
            pltpu.SemaphoreType.DMA((2 * N_REL,)),
        ],
    )(inputs, database.reshape(2 * N_REL, N_NODE // 2, N_NODE), arg1_weights, arg2_weights, op_weights, chain_weights)
